# Initial kernel scaffold; baseline (speedup 1.0000x reference)
import jax
import jax.numpy as jnp
from jax.experimental import pallas as pl


def kernel(query_rel, query_time, src_idx_0, rel_0, batch_idx_0, edge_time_0, dst_idx_0, src_idx_1, rel_1, batch_idx_1, edge_time_1, dst_idx_1, src_idx_2, rel_2, batch_idx_2, edge_time_2, dst_idx_2, final_batch, final_ent, rela_embed_0, W1_0, W2_0, rela_embed_1, W1_1, W2_1, rela_embed_2, W1_2, W2_2, time_embed, W_past, W_now, W_future, W_cls, b_cls):
    raise NotImplementedError("write your pallas kernel here")


# speedup vs baseline: 2.0613x; 2.0613x over previous
import functools
import jax
import jax.numpy as jnp
from jax import lax
from jax.experimental import pallas as pl
from jax.experimental.pallas import tpu as pltpu
from jax.experimental.pallas import tpu_sc as plsc

N_ENT = 10000
N_REL = 231
N_TIME = 365
H = 64
ATTN = 5
B = 8
E = 320000

NC = 2
NS = 16
NW = NC * NS
EPW = E // NW
C0 = 400
CN = 80
ZR = 1000
NZ = N_ENT // ZR

_mesh = plsc.VectorSubcoreMesh(core_axis_name="c", subcore_axis_name="s")
_params = pltpu.CompilerParams(needs_layout_passes=False,
                               use_tc_tiling_on_sc=False)


def _sigmoid16(x):
    return 1.0 / (1.0 + jnp.exp(-x))


def _dt_parts(etv, dstv, qt_t, bidx16, g):
    e16 = etv[pl.ds(g * 16, 16)]
    d16 = dstv[pl.ds(g * 16, 16)]
    qt16 = plsc.load_gather(qt_t, [bidx16])
    dt = e16 - qt16
    adt = jnp.abs(dt)
    cls = jnp.where(dt > 0, 0, jnp.where(dt < 0, 2, 1))
    return adt, cls, d16


def _zero_acc(acc_sh, zrows, sid):
    @pl.when(sid < NZ)
    def _():
        pltpu.sync_copy(zrows, acc_sh.at[pl.ds(sid * ZR, ZR)])


def _dump_acc(acc_sh, out_hbm, cid, s, sid):
    @pl.when(sid < NZ)
    def _():
        pltpu.sync_copy(acc_sh.at[pl.ds(sid * ZR, ZR)],
                        out_hbm.at[cid, s, pl.ds(sid * ZR, ZR)])


@functools.partial(
    pl.kernel,
    out_type=jax.ShapeDtypeStruct((NC, 3, N_ENT, H), jnp.float32),
    mesh=_mesh,
    compiler_params=_params,
    scratch_types=[
        pltpu.VMEM((C0,), jnp.int32),
        pltpu.VMEM((C0,), jnp.int32),
        pltpu.VMEM((C0,), jnp.int32),
        pltpu.VMEM((C0,), jnp.int32),
        pltpu.VMEM((C0,), jnp.int32),
        pltpu.VMEM((C0,), jnp.float32),
        pltpu.VMEM((C0, H), jnp.float32),
        pltpu.VMEM((N_REL, H), jnp.float32),
        pltpu.VMEM((N_TIME, H), jnp.float32),
        pltpu.VMEM((N_REL * B,), jnp.float32),
        pltpu.VMEM((128,), jnp.int32),
        pltpu.VMEM_SHARED((N_ENT, H), jnp.float32),
    ],
)
def _sc_layer0(rel_hbm, bidx_hbm, et_hbm, dst_hbm, rela_hbm, time_hbm,
               s0_hbm, qt_hbm, zrows_hbm, out_hbm,
               relv, bv, etv, dstv, adtv, scorev, msg,
               rela_t, time_t, s0_t, qt_t, acc_sh):
    cid = lax.axis_index("c")
    sid = lax.axis_index("s")
    base = (cid * NS + sid) * EPW

    pltpu.sync_copy(rela_hbm, rela_t)
    pltpu.sync_copy(time_hbm, time_t)
    pltpu.sync_copy(s0_hbm, s0_t)
    pltpu.sync_copy(qt_hbm, qt_t)

    for s in range(3):
        _zero_acc(acc_sh, zrows_hbm, sid)
        plsc.subcore_barrier()

        def chunk_body(t, _):
            off = base + t * C0
            pltpu.sync_copy(rel_hbm.at[pl.ds(off, C0)], relv)
            pltpu.sync_copy(bidx_hbm.at[pl.ds(off, C0)], bv)
            pltpu.sync_copy(et_hbm.at[pl.ds(off, C0)], etv)
            pltpu.sync_copy(dst_hbm.at[pl.ds(off, C0)], dstv)

            def group_body(g, _):
                r16 = relv[pl.ds(g * 16, 16)]
                b16 = bv[pl.ds(g * 16, 16)]
                adt, cls, _d = _dt_parts(etv, dstv, qt_t, b16, g)
                adtv[pl.ds(g * 16, 16)] = adt
                sc = plsc.load_gather(s0_t, [r16 * B + b16])
                scorev[pl.ds(g * 16, 16)] = jnp.where(cls == s, sc, 0.0)
                return 0

            lax.fori_loop(0, C0 // 16, group_body, 0)

            def edge_body(g, _):
                sv = scorev[pl.ds(g * 16, 16)]
                rv = relv[pl.ds(g * 16, 16)]
                av = adtv[pl.ds(g * 16, 16)]
                for lane in range(16):
                    e = g * 16 + lane
                    sl_, re, ae = sv[lane], rv[lane], av[lane]
                    for k in range(4):
                        sl = pl.ds(k * 16, 16)
                        msg[e, sl] = sl_ * (rela_t[re, sl] + time_t[ae, sl])
                return 0

            lax.fori_loop(0, C0 // 16, edge_body, 0)
            pltpu.sync_copy(msg, acc_sh.at[dstv], add=True)
            return 0

        lax.fori_loop(0, EPW // C0, chunk_body, 0)
        plsc.subcore_barrier()
        _dump_acc(acc_sh, out_hbm, cid, s, sid)
        plsc.subcore_barrier()


@functools.partial(
    pl.kernel,
    out_type=jax.ShapeDtypeStruct((NC, 3, N_ENT, H), jnp.float32),
    mesh=_mesh,
    compiler_params=_params,
    scratch_types=[
        pltpu.VMEM((CN,), jnp.int32),
        pltpu.VMEM((CN,), jnp.int32),
        pltpu.VMEM((CN,), jnp.int32),
        pltpu.VMEM((CN,), jnp.int32),
        pltpu.VMEM((CN,), jnp.int32),
        pltpu.VMEM((CN,), jnp.int32),
        pltpu.VMEM((CN,), jnp.float32),
        pltpu.VMEM((CN, H), jnp.float32),
        pltpu.VMEM((CN, H), jnp.float32),
        pltpu.VMEM((N_TIME, H), jnp.float32),
        pltpu.VMEM((N_ENT * ATTN,), jnp.float32),
        pltpu.VMEM((N_REL * ATTN,), jnp.float32),
        pltpu.VMEM((B * ATTN,), jnp.float32),
        pltpu.VMEM((128,), jnp.float32),
        pltpu.VMEM((128,), jnp.int32),
        pltpu.SemaphoreType.DMA,
        pltpu.SemaphoreType.DMA,
        pltpu.VMEM_SHARED((N_ENT, H), jnp.float32),
    ],
)
def _sc_layer(src_hbm, rel_hbm, bidx_hbm, et_hbm, dst_hbm, hid_hbm,
              rela_hbm, time_hbm, hf_hbm, rf_hbm, qf_hbm, w2_hbm, qt_hbm,
              zrows_hbm, out_hbm,
              srcv, relv, bv, etv, dstv, adtv, scorev, hrows, rrows,
              time_t, hf_t, rf_t, qf_t, w2_t, qt_t, hsem, rsem, acc_sh):
    cid = lax.axis_index("c")
    sid = lax.axis_index("s")
    base = (cid * NS + sid) * EPW

    pltpu.sync_copy(time_hbm, time_t)
    pltpu.sync_copy(hf_hbm, hf_t)
    pltpu.sync_copy(rf_hbm, rf_t)
    pltpu.sync_copy(qf_hbm, qf_t)
    pltpu.sync_copy(w2_hbm, w2_t)
    pltpu.sync_copy(qt_hbm, qt_t)

    for s in range(3):
        _zero_acc(acc_sh, zrows_hbm, sid)
        plsc.subcore_barrier()

        def chunk_body(t, _):
            off = base + t * CN
            pltpu.sync_copy(src_hbm.at[pl.ds(off, CN)], srcv)
            pltpu.sync_copy(rel_hbm.at[pl.ds(off, CN)], relv)
            hg = pltpu.async_copy(hid_hbm.at[srcv], hrows, hsem)
            rg = pltpu.async_copy(rela_hbm.at[relv], rrows, rsem)
            pltpu.sync_copy(bidx_hbm.at[pl.ds(off, CN)], bv)
            pltpu.sync_copy(et_hbm.at[pl.ds(off, CN)], etv)
            pltpu.sync_copy(dst_hbm.at[pl.ds(off, CN)], dstv)
            w2v = w2_t[pl.ds(0, 16)]

            def group_body(g, _):
                s16 = srcv[pl.ds(g * 16, 16)]
                r16 = relv[pl.ds(g * 16, 16)]
                b16 = bv[pl.ds(g * 16, 16)]
                adt, cls, _d = _dt_parts(etv, dstv, qt_t, b16, g)
                adtv[pl.ds(g * 16, 16)] = adt
                sc_acc = jnp.zeros((16,), jnp.float32)
                for j in range(ATTN):
                    jj = jnp.full((16,), j, jnp.int32)
                    fj = (plsc.load_gather(hf_t, [s16 * ATTN + jj])
                          + plsc.load_gather(rf_t, [r16 * ATTN + jj])
                          + plsc.load_gather(qf_t, [b16 * ATTN + jj]))
                    sc_acc = sc_acc + jnp.maximum(fj, 0.0) * w2v[j]
                sc = _sigmoid16(sc_acc)
                scorev[pl.ds(g * 16, 16)] = jnp.where(cls == s, sc, 0.0)
                return 0

            lax.fori_loop(0, CN // 16, group_body, 0)
            hg.wait()
            rg.wait()

            def edge_body(g, _):
                sv = scorev[pl.ds(g * 16, 16)]
                av = adtv[pl.ds(g * 16, 16)]
                for lane in range(16):
                    e = g * 16 + lane
                    sl_, ae = sv[lane], av[lane]
                    for k in range(4):
                        sl = pl.ds(k * 16, 16)
                        hrows[e, sl] = sl_ * (hrows[e, sl] + rrows[e, sl]
                                              + time_t[ae, sl])
                return 0

            lax.fori_loop(0, CN // 16, edge_body, 0)
            pltpu.sync_copy(hrows, acc_sh.at[dstv], add=True)
            return 0

        lax.fori_loop(0, EPW // CN, chunk_body, 0)
        plsc.subcore_barrier()
        _dump_acc(acc_sh, out_hbm, cid, s, sid)
        plsc.subcore_barrier()


def _combine_hidden(parts_ref, wf_ref, wn_ref, wp_ref):
    p = parts_ref[0] + parts_ref[1]
    dn = (((1,), (1,)), ((), ()))
    hp = lax.Precision.HIGHEST
    hid = (lax.dot_general(p[0], wf_ref[...], dn, precision=hp)
           + lax.dot_general(p[1], wn_ref[...], dn, precision=hp)
           + lax.dot_general(p[2], wp_ref[...], dn, precision=hp))
    return jnp.maximum(hid, 0.0)


def _tc_mid_body(parts_ref, wf_ref, wn_ref, wp_ref, w1a_ref,
                 hid_ref, hf_ref):
    hid = _combine_hidden(parts_ref, wf_ref, wn_ref, wp_ref)
    hid_ref[...] = hid
    hf_ref[...] = lax.dot_general(hid, w1a_ref[...], (((1,), (1,)), ((), ())),
                                 precision=lax.Precision.HIGHEST)


def _tc_final_body(parts_ref, wf_ref, wn_ref, wp_ref, wcls_ref, bcls_ref,
                   res_ref):
    hid = _combine_hidden(parts_ref, wf_ref, wn_ref, wp_ref)
    res_ref[...] = (lax.dot_general(hid, wcls_ref[...],
                                    (((1,), (1,)), ((), ())),
                                    precision=lax.Precision.HIGHEST)
                    + bcls_ref[0, 0])


_TCB = 2000
_parts_spec = pl.BlockSpec((NC, 3, _TCB, H), lambda i: (0, 0, i, 0))
_w_spec = pl.BlockSpec((H, H), lambda i: (0, 0))

_tc_mid = pl.pallas_call(
    _tc_mid_body,
    grid=(N_ENT // _TCB,),
    in_specs=[_parts_spec, _w_spec, _w_spec, _w_spec,
              pl.BlockSpec((ATTN, H), lambda i: (0, 0))],
    out_specs=[pl.BlockSpec((_TCB, H), lambda i: (i, 0)),
               pl.BlockSpec((_TCB, ATTN), lambda i: (i, 0))],
    out_shape=[jax.ShapeDtypeStruct((N_ENT, H), jnp.float32),
               jax.ShapeDtypeStruct((N_ENT, ATTN), jnp.float32)],
)

_tc_final = pl.pallas_call(
    _tc_final_body,
    grid=(N_ENT // _TCB,),
    in_specs=[_parts_spec, _w_spec, _w_spec, _w_spec,
              pl.BlockSpec((8, H), lambda i: (0, 0)),
              pl.BlockSpec((1, 1), lambda i: (0, 0))],
    out_specs=pl.BlockSpec((_TCB, 8), lambda i: (i, 0)),
    out_shape=jax.ShapeDtypeStruct((N_ENT, 8), jnp.float32),
)


def _tc_prep_body(qr_ref, r0_ref, r1_ref, r2_ref, w10_ref, w11_ref, w12_ref,
                  w20_ref, s0_ref, rf1_ref, qf1_ref, rf2_ref, qf2_ref):
    qr = qr_ref[...]
    oh = jnp.where(jax.lax.broadcasted_iota(jnp.int32, (B, N_REL), 1) == qr,
                   1.0, 0.0)
    dn = (((1,), (1,)), ((), ()))
    hp = lax.Precision.HIGHEST
    outs_rf = [None, rf1_ref, rf2_ref]
    outs_qf = [None, qf1_ref, qf2_ref]
    for l, (r_ref, w1_ref) in enumerate([(r0_ref, w10_ref),
                                         (r1_ref, w11_ref),
                                         (r2_ref, w12_ref)]):
        rela = r_ref[...]
        w1 = w1_ref[...]
        rf = lax.dot_general(rela, w1[:, H:2 * H], dn, precision=hp)
        qrow = jnp.dot(oh, rela, precision=lax.Precision.HIGHEST)
        qf = lax.dot_general(qrow, w1[:, 2 * H:], dn, precision=hp)
        if l == 0:
            w2 = w20_ref[...]
            sacc = jnp.zeros((N_REL, B), jnp.float32)
            for j in range(ATTN):
                f = rf[:, j:j + 1] + qf[:, j:j + 1].reshape(1, B)
                sacc = sacc + jnp.maximum(f, 0.0) * w2[0, j]
            s0_ref[...] = 1.0 / (1.0 + jnp.exp(-sacc))
        else:
            outs_rf[l][...] = rf
            outs_qf[l][...] = qf


_tc_prep = pl.pallas_call(
    _tc_prep_body,
    out_shape=[jax.ShapeDtypeStruct((N_REL, B), jnp.float32),
               jax.ShapeDtypeStruct((N_REL, ATTN), jnp.float32),
               jax.ShapeDtypeStruct((B, ATTN), jnp.float32),
               jax.ShapeDtypeStruct((N_REL, ATTN), jnp.float32),
               jax.ShapeDtypeStruct((B, ATTN), jnp.float32)],
)


def kernel(query_rel, query_time, src_idx_0, rel_0, batch_idx_0, edge_time_0,
           dst_idx_0, src_idx_1, rel_1, batch_idx_1, edge_time_1, dst_idx_1,
           src_idx_2, rel_2, batch_idx_2, edge_time_2, dst_idx_2,
           final_batch, final_ent, rela_embed_0, W1_0, W2_0, rela_embed_1,
           W1_1, W2_1, rela_embed_2, W1_2, W2_2, time_embed, W_past, W_now,
           W_future, W_cls, b_cls):
    f32 = jnp.float32
    i32 = jnp.int32
    qt_pad = jnp.zeros((128,), i32).at[:B].set(query_time.astype(i32))
    zrows = jnp.zeros((ZR, H), f32)

    s0, rf1, qf1, rf2, qf2 = _tc_prep(
        query_rel.astype(i32).reshape(B, 1), rela_embed_0, rela_embed_1,
        rela_embed_2, W1_0, W1_1, W1_2, W2_0)

    parts0 = _sc_layer0(
        rel_0.astype(i32), batch_idx_0.astype(i32), edge_time_0.astype(i32),
        dst_idx_0.astype(i32), rela_embed_0, time_embed,
        s0.reshape(-1), qt_pad, zrows)

    hid1, hf1 = _tc_mid(parts0, W_future, W_now, W_past, W1_1[:, :H])

    w2p1 = jnp.zeros((128,), f32).at[:ATTN].set(W2_1[0])
    parts1 = _sc_layer(
        src_idx_1.astype(i32), rel_1.astype(i32), batch_idx_1.astype(i32),
        edge_time_1.astype(i32), dst_idx_1.astype(i32), hid1,
        rela_embed_1, time_embed, hf1.reshape(-1), rf1.reshape(-1),
        qf1.reshape(-1), w2p1, qt_pad, zrows)

    hid2, hf2 = _tc_mid(parts1, W_future, W_now, W_past, W1_2[:, :H])

    w2p2 = jnp.zeros((128,), f32).at[:ATTN].set(W2_2[0])
    parts2 = _sc_layer(
        src_idx_2.astype(i32), rel_2.astype(i32), batch_idx_2.astype(i32),
        edge_time_2.astype(i32), dst_idx_2.astype(i32), hid2,
        rela_embed_2, time_embed, hf2.reshape(-1), rf2.reshape(-1),
        qf2.reshape(-1), w2p2, qt_pad, zrows)

    wcls_pad = jnp.zeros((8, H), f32).at[0].set(W_cls[0])
    res = _tc_final(parts2, W_future, W_now, W_past, wcls_pad,
                    b_cls.reshape(1, 1))[:, 0]

    return (jnp.zeros((B, N_ENT), f32)
            .at[final_batch, final_ent].set(res))


# speedup vs baseline: 2.0724x; 1.0054x over previous
import functools
import jax
import jax.numpy as jnp
from jax import lax
from jax.experimental import pallas as pl
from jax.experimental.pallas import tpu as pltpu
from jax.experimental.pallas import tpu_sc as plsc

N_ENT = 10000
N_REL = 231
N_TIME = 365
H = 64
ATTN = 5
B = 8
E = 320000

NC = 2
NS = 16
NW = NC * NS
EPW = E // NW
C0 = 400
CN = 80
ZR = 1000
NZ = N_ENT // ZR

_mesh = plsc.VectorSubcoreMesh(core_axis_name="c", subcore_axis_name="s")
_params = pltpu.CompilerParams(needs_layout_passes=False,
                               use_tc_tiling_on_sc=False)


def _sigmoid16(x):
    return 1.0 / (1.0 + jnp.exp(-x))


def _dt_parts(etv, dstv, qt_t, bidx16, g):
    e16 = etv[pl.ds(g * 16, 16)]
    d16 = dstv[pl.ds(g * 16, 16)]
    qt16 = plsc.load_gather(qt_t, [bidx16])
    dt = e16 - qt16
    adt = jnp.abs(dt)
    cls = jnp.where(dt > 0, 0, jnp.where(dt < 0, 2, 1))
    return adt, cls, d16


def _zero_acc(acc_sh, zrows, sid):
    @pl.when(sid < NZ)
    def _():
        pltpu.sync_copy(zrows, acc_sh.at[pl.ds(sid * ZR, ZR)])


def _dump_acc(acc_sh, out_hbm, cid, s, sid):
    @pl.when(sid < NZ)
    def _():
        pltpu.sync_copy(acc_sh.at[pl.ds(sid * ZR, ZR)],
                        out_hbm.at[cid, s, pl.ds(sid * ZR, ZR)])


@functools.partial(
    pl.kernel,
    out_type=jax.ShapeDtypeStruct((NC, 3, N_ENT, H), jnp.float32),
    mesh=_mesh,
    compiler_params=_params,
    scratch_types=[
        pltpu.VMEM((C0,), jnp.int32),
        pltpu.VMEM((C0,), jnp.int32),
        pltpu.VMEM((C0,), jnp.int32),
        pltpu.VMEM((C0,), jnp.int32),
        pltpu.VMEM((C0,), jnp.int32),
        pltpu.VMEM((C0,), jnp.float32),
        pltpu.VMEM((C0, H), jnp.float32),
        pltpu.VMEM((N_REL, H), jnp.float32),
        pltpu.VMEM((N_TIME, H), jnp.float32),
        pltpu.VMEM((N_REL * B,), jnp.float32),
        pltpu.VMEM((128,), jnp.int32),
        pltpu.VMEM_SHARED((N_ENT, H), jnp.float32),
    ],
)
def _sc_layer0(rel_hbm, bidx_hbm, et_hbm, dst_hbm, rela_hbm, time_hbm,
               s0_hbm, qt_hbm, zrows_hbm, out_hbm,
               relv, bv, etv, dstv, adtv, scorev, msg,
               rela_t, time_t, s0_t, qt_t, acc_sh):
    cid = lax.axis_index("c")
    sid = lax.axis_index("s")
    base = (cid * NS + sid) * EPW

    pltpu.sync_copy(rela_hbm, rela_t)
    pltpu.sync_copy(time_hbm, time_t)
    pltpu.sync_copy(s0_hbm, s0_t)
    pltpu.sync_copy(qt_hbm, qt_t)

    for s in range(3):
        _zero_acc(acc_sh, zrows_hbm, sid)
        plsc.subcore_barrier()

        def chunk_body(t, _):
            off = base + t * C0
            pltpu.sync_copy(rel_hbm.at[pl.ds(off, C0)], relv)
            pltpu.sync_copy(bidx_hbm.at[pl.ds(off, C0)], bv)
            pltpu.sync_copy(et_hbm.at[pl.ds(off, C0)], etv)
            pltpu.sync_copy(dst_hbm.at[pl.ds(off, C0)], dstv)

            def group_body(g, cnt):
                r16 = relv[pl.ds(g * 16, 16)]
                b16 = bv[pl.ds(g * 16, 16)]
                adt, cls, _d = _dt_parts(etv, dstv, qt_t, b16, g)
                adtv[pl.ds(g * 16, 16)] = adt
                sc = plsc.load_gather(s0_t, [r16 * B + b16])
                m = cls == s
                scorev[pl.ds(g * 16, 16)] = jnp.where(m, sc, 0.0)
                return cnt + jnp.max(jnp.where(m, 1, 0))

            nhit = lax.fori_loop(0, C0 // 16, group_body, 0)

            def edge_body(g, _):
                sv = scorev[pl.ds(g * 16, 16)]
                rv = relv[pl.ds(g * 16, 16)]
                av = adtv[pl.ds(g * 16, 16)]
                for lane in range(16):
                    e = g * 16 + lane
                    sl_, re, ae = sv[lane], rv[lane], av[lane]
                    for k in range(4):
                        sl = pl.ds(k * 16, 16)
                        msg[e, sl] = sl_ * (rela_t[re, sl] + time_t[ae, sl])
                return 0

            @pl.when(nhit > 0)
            def _():
                lax.fori_loop(0, C0 // 16, edge_body, 0)
                pltpu.sync_copy(msg, acc_sh.at[dstv], add=True)
            return 0

        lax.fori_loop(0, EPW // C0, chunk_body, 0)
        plsc.subcore_barrier()
        _dump_acc(acc_sh, out_hbm, cid, s, sid)
        plsc.subcore_barrier()


@functools.partial(
    pl.kernel,
    out_type=jax.ShapeDtypeStruct((NC, 3, N_ENT, H), jnp.float32),
    mesh=_mesh,
    compiler_params=_params,
    scratch_types=[
        pltpu.VMEM((CN,), jnp.int32),
        pltpu.VMEM((CN,), jnp.int32),
        pltpu.VMEM((CN,), jnp.int32),
        pltpu.VMEM((CN,), jnp.int32),
        pltpu.VMEM((CN,), jnp.int32),
        pltpu.VMEM((CN,), jnp.int32),
        pltpu.VMEM((CN,), jnp.float32),
        pltpu.VMEM((CN, H), jnp.float32),
        pltpu.VMEM((CN, H), jnp.float32),
        pltpu.VMEM((N_TIME, H), jnp.float32),
        pltpu.VMEM((N_ENT * ATTN,), jnp.float32),
        pltpu.VMEM((N_REL * ATTN,), jnp.float32),
        pltpu.VMEM((B * ATTN,), jnp.float32),
        pltpu.VMEM((128,), jnp.float32),
        pltpu.VMEM((128,), jnp.int32),
        pltpu.SemaphoreType.DMA,
        pltpu.SemaphoreType.DMA,
        pltpu.VMEM_SHARED((N_ENT, H), jnp.float32),
    ],
)
def _sc_layer(src_hbm, rel_hbm, bidx_hbm, et_hbm, dst_hbm, hid_hbm,
              rela_hbm, time_hbm, hf_hbm, rf_hbm, qf_hbm, w2_hbm, qt_hbm,
              zrows_hbm, out_hbm,
              srcv, relv, bv, etv, dstv, adtv, scorev, hrows, rrows,
              time_t, hf_t, rf_t, qf_t, w2_t, qt_t, hsem, rsem, acc_sh):
    cid = lax.axis_index("c")
    sid = lax.axis_index("s")
    base = (cid * NS + sid) * EPW

    pltpu.sync_copy(time_hbm, time_t)
    pltpu.sync_copy(hf_hbm, hf_t)
    pltpu.sync_copy(rf_hbm, rf_t)
    pltpu.sync_copy(qf_hbm, qf_t)
    pltpu.sync_copy(w2_hbm, w2_t)
    pltpu.sync_copy(qt_hbm, qt_t)

    for s in range(3):
        _zero_acc(acc_sh, zrows_hbm, sid)
        plsc.subcore_barrier()

        def chunk_body(t, _):
            off = base + t * CN
            pltpu.sync_copy(src_hbm.at[pl.ds(off, CN)], srcv)
            pltpu.sync_copy(rel_hbm.at[pl.ds(off, CN)], relv)
            pltpu.sync_copy(bidx_hbm.at[pl.ds(off, CN)], bv)
            pltpu.sync_copy(et_hbm.at[pl.ds(off, CN)], etv)
            pltpu.sync_copy(dst_hbm.at[pl.ds(off, CN)], dstv)
            w2v = w2_t[pl.ds(0, 16)]

            def group_body(g, cnt):
                s16 = srcv[pl.ds(g * 16, 16)]
                r16 = relv[pl.ds(g * 16, 16)]
                b16 = bv[pl.ds(g * 16, 16)]
                adt, cls, _d = _dt_parts(etv, dstv, qt_t, b16, g)
                adtv[pl.ds(g * 16, 16)] = adt
                sc_acc = jnp.zeros((16,), jnp.float32)
                for j in range(ATTN):
                    jj = jnp.full((16,), j, jnp.int32)
                    fj = (plsc.load_gather(hf_t, [s16 * ATTN + jj])
                          + plsc.load_gather(rf_t, [r16 * ATTN + jj])
                          + plsc.load_gather(qf_t, [b16 * ATTN + jj]))
                    sc_acc = sc_acc + jnp.maximum(fj, 0.0) * w2v[j]
                sc = _sigmoid16(sc_acc)
                m = cls == s
                scorev[pl.ds(g * 16, 16)] = jnp.where(m, sc, 0.0)
                return cnt + jnp.max(jnp.where(m, 1, 0))

            nhit = lax.fori_loop(0, CN // 16, group_body, 0)

            def edge_body(g, _):
                sv = scorev[pl.ds(g * 16, 16)]
                av = adtv[pl.ds(g * 16, 16)]
                for lane in range(16):
                    e = g * 16 + lane
                    sl_, ae = sv[lane], av[lane]
                    for k in range(4):
                        sl = pl.ds(k * 16, 16)
                        hrows[e, sl] = sl_ * (hrows[e, sl] + rrows[e, sl]
                                              + time_t[ae, sl])
                return 0

            @pl.when(nhit > 0)
            def _():
                hg = pltpu.async_copy(hid_hbm.at[srcv], hrows, hsem)
                rg = pltpu.async_copy(rela_hbm.at[relv], rrows, rsem)
                hg.wait()
                rg.wait()
                lax.fori_loop(0, CN // 16, edge_body, 0)
                pltpu.sync_copy(hrows, acc_sh.at[dstv], add=True)
            return 0

        lax.fori_loop(0, EPW // CN, chunk_body, 0)
        plsc.subcore_barrier()
        _dump_acc(acc_sh, out_hbm, cid, s, sid)
        plsc.subcore_barrier()


def _combine_hidden(parts_ref, wf_ref, wn_ref, wp_ref):
    p = parts_ref[0] + parts_ref[1]
    dn = (((1,), (1,)), ((), ()))
    hp = lax.Precision.HIGHEST
    hid = (lax.dot_general(p[0], wf_ref[...], dn, precision=hp)
           + lax.dot_general(p[1], wn_ref[...], dn, precision=hp)
           + lax.dot_general(p[2], wp_ref[...], dn, precision=hp))
    return jnp.maximum(hid, 0.0)


def _tc_mid_body(parts_ref, wf_ref, wn_ref, wp_ref, w1a_ref,
                 hid_ref, hf_ref):
    hid = _combine_hidden(parts_ref, wf_ref, wn_ref, wp_ref)
    hid_ref[...] = hid
    hf_ref[...] = lax.dot_general(hid, w1a_ref[...], (((1,), (1,)), ((), ())),
                                 precision=lax.Precision.HIGHEST)


def _tc_final_body(parts_ref, wf_ref, wn_ref, wp_ref, wcls_ref, bcls_ref,
                   res_ref):
    hid = _combine_hidden(parts_ref, wf_ref, wn_ref, wp_ref)
    res_ref[...] = (lax.dot_general(hid, wcls_ref[...],
                                    (((1,), (1,)), ((), ())),
                                    precision=lax.Precision.HIGHEST)
                    + bcls_ref[0, 0])


_TCB = 2000
_parts_spec = pl.BlockSpec((NC, 3, _TCB, H), lambda i: (0, 0, i, 0))
_w_spec = pl.BlockSpec((H, H), lambda i: (0, 0))

_tc_mid = pl.pallas_call(
    _tc_mid_body,
    grid=(N_ENT // _TCB,),
    in_specs=[_parts_spec, _w_spec, _w_spec, _w_spec,
              pl.BlockSpec((ATTN, H), lambda i: (0, 0))],
    out_specs=[pl.BlockSpec((_TCB, H), lambda i: (i, 0)),
               pl.BlockSpec((_TCB, ATTN), lambda i: (i, 0))],
    out_shape=[jax.ShapeDtypeStruct((N_ENT, H), jnp.float32),
               jax.ShapeDtypeStruct((N_ENT, ATTN), jnp.float32)],
)

_tc_final = pl.pallas_call(
    _tc_final_body,
    grid=(N_ENT // _TCB,),
    in_specs=[_parts_spec, _w_spec, _w_spec, _w_spec,
              pl.BlockSpec((8, H), lambda i: (0, 0)),
              pl.BlockSpec((1, 1), lambda i: (0, 0))],
    out_specs=pl.BlockSpec((_TCB, 8), lambda i: (i, 0)),
    out_shape=jax.ShapeDtypeStruct((N_ENT, 8), jnp.float32),
)


def _tc_prep_body(qr_ref, r0_ref, r1_ref, r2_ref, w10_ref, w11_ref, w12_ref,
                  w20_ref, s0_ref, rf1_ref, qf1_ref, rf2_ref, qf2_ref):
    qr = qr_ref[...]
    oh = jnp.where(jax.lax.broadcasted_iota(jnp.int32, (B, N_REL), 1) == qr,
                   1.0, 0.0)
    dn = (((1,), (1,)), ((), ()))
    hp = lax.Precision.HIGHEST
    outs_rf = [None, rf1_ref, rf2_ref]
    outs_qf = [None, qf1_ref, qf2_ref]
    for l, (r_ref, w1_ref) in enumerate([(r0_ref, w10_ref),
                                         (r1_ref, w11_ref),
                                         (r2_ref, w12_ref)]):
        rela = r_ref[...]
        w1 = w1_ref[...]
        rf = lax.dot_general(rela, w1[:, H:2 * H], dn, precision=hp)
        qrow = jnp.dot(oh, rela, precision=lax.Precision.HIGHEST)
        qf = lax.dot_general(qrow, w1[:, 2 * H:], dn, precision=hp)
        if l == 0:
            w2 = w20_ref[...]
            sacc = jnp.zeros((N_REL, B), jnp.float32)
            for j in range(ATTN):
                f = rf[:, j:j + 1] + qf[:, j:j + 1].reshape(1, B)
                sacc = sacc + jnp.maximum(f, 0.0) * w2[0, j]
            s0_ref[...] = 1.0 / (1.0 + jnp.exp(-sacc))
        else:
            outs_rf[l][...] = rf
            outs_qf[l][...] = qf


_tc_prep = pl.pallas_call(
    _tc_prep_body,
    out_shape=[jax.ShapeDtypeStruct((N_REL, B), jnp.float32),
               jax.ShapeDtypeStruct((N_REL, ATTN), jnp.float32),
               jax.ShapeDtypeStruct((B, ATTN), jnp.float32),
               jax.ShapeDtypeStruct((N_REL, ATTN), jnp.float32),
               jax.ShapeDtypeStruct((B, ATTN), jnp.float32)],
)


def kernel(query_rel, query_time, src_idx_0, rel_0, batch_idx_0, edge_time_0,
           dst_idx_0, src_idx_1, rel_1, batch_idx_1, edge_time_1, dst_idx_1,
           src_idx_2, rel_2, batch_idx_2, edge_time_2, dst_idx_2,
           final_batch, final_ent, rela_embed_0, W1_0, W2_0, rela_embed_1,
           W1_1, W2_1, rela_embed_2, W1_2, W2_2, time_embed, W_past, W_now,
           W_future, W_cls, b_cls):
    f32 = jnp.float32
    i32 = jnp.int32
    qt_pad = jnp.zeros((128,), i32).at[:B].set(query_time.astype(i32))
    zrows = jnp.zeros((ZR, H), f32)

    s0, rf1, qf1, rf2, qf2 = _tc_prep(
        query_rel.astype(i32).reshape(B, 1), rela_embed_0, rela_embed_1,
        rela_embed_2, W1_0, W1_1, W1_2, W2_0)

    parts0 = _sc_layer0(
        rel_0.astype(i32), batch_idx_0.astype(i32), edge_time_0.astype(i32),
        dst_idx_0.astype(i32), rela_embed_0, time_embed,
        s0.reshape(-1), qt_pad, zrows)

    hid1, hf1 = _tc_mid(parts0, W_future, W_now, W_past, W1_1[:, :H])

    w2p1 = jnp.zeros((128,), f32).at[:ATTN].set(W2_1[0])
    parts1 = _sc_layer(
        src_idx_1.astype(i32), rel_1.astype(i32), batch_idx_1.astype(i32),
        edge_time_1.astype(i32), dst_idx_1.astype(i32), hid1,
        rela_embed_1, time_embed, hf1.reshape(-1), rf1.reshape(-1),
        qf1.reshape(-1), w2p1, qt_pad, zrows)

    hid2, hf2 = _tc_mid(parts1, W_future, W_now, W_past, W1_2[:, :H])

    w2p2 = jnp.zeros((128,), f32).at[:ATTN].set(W2_2[0])
    parts2 = _sc_layer(
        src_idx_2.astype(i32), rel_2.astype(i32), batch_idx_2.astype(i32),
        edge_time_2.astype(i32), dst_idx_2.astype(i32), hid2,
        rela_embed_2, time_embed, hf2.reshape(-1), rf2.reshape(-1),
        qf2.reshape(-1), w2p2, qt_pad, zrows)

    wcls_pad = jnp.zeros((8, H), f32).at[0].set(W_cls[0])
    res = _tc_final(parts2, W_future, W_now, W_past, wcls_pad,
                    b_cls.reshape(1, 1))[:, 0]

    return (jnp.zeros((B, N_ENT), f32)
            .at[final_batch, final_ent].set(res))
